# baseline (speedup 1.0000x reference)
import functools
import math

import jax
import jax.numpy as jnp
from jax.experimental import pallas as pl
from jax.experimental.pallas import tpu as pltpu

_CUTOFF = 1.0


def _ssp(v):
    return (jnp.maximum(v, 0.0)
            + jnp.log(1.0 + jnp.exp(-jnp.abs(v)))
            - math.log(2.0))


def _fused_kernel(x_ref, r_ref, nbr_ref, mask_ref, f_ref,
                  W1_ref, b1_ref, W2_ref, b2_ref, Win_ref,
                  Wout_ref, bout_ref, Wd_ref, bd_ref,
                  o_ref, *, chunk, nb, n_atoms, nf, nsb):
    y = jnp.dot(x_ref[0].astype(jnp.bfloat16),
                Win_ref[...].astype(jnp.bfloat16),
                preferred_element_type=jnp.float32).astype(jnp.bfloat16)
    W1 = W1_ref[...].astype(jnp.bfloat16)
    W2 = W2_ref[...].astype(jnp.bfloat16)

    rows = chunk * nb
    for ch in range(n_atoms // chunk):
        sl = pl.ds(ch * chunk, chunk)
        f = f_ref[0, sl].reshape(rows, nsb).astype(jnp.bfloat16)
        h = _ssp((jnp.dot(f, W1, preferred_element_type=jnp.float32)
                  + b1_ref[...]).astype(jnp.bfloat16))
        wf = (jnp.dot(h, W2, preferred_element_type=jnp.float32)
              + b2_ref[...])

        r = r_ref[0, sl]
        c = (0.5 * (jnp.cos(r * (math.pi / _CUTOFF)) + 1.0)
             * (r < _CUTOFF).astype(jnp.float32) * mask_ref[0, sl])

        idx = nbr_ref[0, sl]
        iota = jax.lax.broadcasted_iota(jnp.int32, (chunk, nb, n_atoms), 2)
        oh = jnp.where(idx[:, :, None] == iota, c[:, :, None], 0.0
                       ).reshape(rows, n_atoms).astype(jnp.bfloat16)
        ynb = jnp.dot(oh, y, preferred_element_type=jnp.float32)

        prod = ynb * wf
        agg = jnp.sum(prod.reshape(chunk, nb, nf), axis=1)

        out = _ssp(jnp.dot(agg, Wout_ref[...],
                           preferred_element_type=jnp.float32)
                   + bout_ref[...])
        o_ref[0, sl] = (jnp.dot(out, Wd_ref[...],
                                preferred_element_type=jnp.float32)
                        + bd_ref[...])


@jax.jit
def kernel(x, r_ij, neighbors, neighbor_mask, f_ij,
           W1, b1, W2, b2, Win, Wout, bout, Wd, bd):
    B, N, NAB = x.shape
    NB = r_ij.shape[2]
    NSB = f_ij.shape[3]
    NF = W1.shape[1]
    CHUNK = 32

    b1r = b1.reshape(1, NF)
    b2r = b2.reshape(1, NF)
    boutr = bout.reshape(1, NAB)
    bdr = bd.reshape(1, NAB)
    nbrs = neighbors.astype(jnp.int32)

    grid = (B,)
    full2d = lambda b: (0, 0)
    blk3 = lambda b: (b, 0, 0)

    out = pl.pallas_call(
        functools.partial(_fused_kernel, chunk=CHUNK, nb=NB,
                          n_atoms=N, nf=NF, nsb=NSB),
        grid=grid,
        in_specs=[
            pl.BlockSpec((1, N, NAB), blk3),
            pl.BlockSpec((1, N, NB), blk3),
            pl.BlockSpec((1, N, NB), blk3),
            pl.BlockSpec((1, N, NB), blk3),
            pl.BlockSpec((1, N, NB, NSB), lambda b: (b, 0, 0, 0)),
            pl.BlockSpec((NSB, NF), full2d),
            pl.BlockSpec((1, NF), full2d),
            pl.BlockSpec((NF, NF), full2d),
            pl.BlockSpec((1, NF), full2d),
            pl.BlockSpec((NAB, NF), full2d),
            pl.BlockSpec((NF, NAB), full2d),
            pl.BlockSpec((1, NAB), full2d),
            pl.BlockSpec((NAB, NAB), full2d),
            pl.BlockSpec((1, NAB), full2d),
        ],
        out_specs=pl.BlockSpec((1, N, NAB), blk3),
        out_shape=jax.ShapeDtypeStruct((B, N, NAB), jnp.float32),
        compiler_params=pltpu.CompilerParams(
            dimension_semantics=("arbitrary",)),
    )(x, r_ij, nbrs, neighbor_mask, f_ij,
      W1, b1r, W2, b2r, Win, Wout, boutr, Wd, bdr)
    return out


# speedup vs baseline: 1.3441x; 1.3441x over previous
import functools
import math

import jax
import jax.numpy as jnp
from jax.experimental import pallas as pl
from jax.experimental.pallas import tpu as pltpu

_CUTOFF = 1.0


def _ssp(v):
    return (jnp.maximum(v, 0.0)
            + jnp.log(0.5 + 0.5 * jnp.exp(-jnp.abs(v))))


def _fused_kernel(x_ref, r_ref, nbr_ref, mask_ref, f_ref,
                  W1_ref, b1_ref, W2_ref, b2_ref, Win_ref,
                  Wout_ref, bout_ref, Wd_ref, bd_ref,
                  o_ref, *, blk_n, nb, n_atoms, nf):
    y = jnp.dot(x_ref[0].astype(jnp.bfloat16),
                Win_ref[...].astype(jnp.bfloat16),
                preferred_element_type=jnp.float32).astype(jnp.bfloat16)

    rows = blk_n * nb
    f = f_ref[0].reshape(rows, f_ref.shape[-1]).astype(jnp.bfloat16)
    h = _ssp((jnp.dot(f, W1_ref[...].astype(jnp.bfloat16),
                      preferred_element_type=jnp.float32)
              + b1_ref[...]).astype(jnp.bfloat16))
    wf = (jnp.dot(h, W2_ref[...].astype(jnp.bfloat16),
                  preferred_element_type=jnp.float32)
          + b2_ref[...])

    r = r_ref[0]
    z = (0.5 - r * (1.0 / _CUTOFF)) * math.pi
    z2 = z * z
    sinz = z * (1.0 + z2 * (-1.0 / 6.0 + z2 * (1.0 / 120.0 + z2 *
                (-1.0 / 5040.0 + z2 * (1.0 / 362880.0)))))
    c = ((0.5 + 0.5 * sinz)
         * (r < _CUTOFF).astype(jnp.float32) * mask_ref[0])

    idx = nbr_ref[0]
    iota = jax.lax.broadcasted_iota(jnp.int32, (blk_n, nb, n_atoms), 2)
    oh = jnp.where(idx[:, :, None] == iota, c[:, :, None], 0.0
                   ).reshape(rows, n_atoms).astype(jnp.bfloat16)
    ynb = jnp.dot(oh, y, preferred_element_type=jnp.float32)

    prod = ynb * wf
    agg = jnp.sum(prod.reshape(blk_n, nb, nf), axis=1)

    out = _ssp(jnp.dot(agg, Wout_ref[...], preferred_element_type=jnp.float32)
               + bout_ref[...])
    o_ref[0] = (jnp.dot(out, Wd_ref[...], preferred_element_type=jnp.float32)
                + bd_ref[...])


@jax.jit
def kernel(x, r_ij, neighbors, neighbor_mask, f_ij,
           W1, b1, W2, b2, Win, Wout, bout, Wd, bd):
    B, N, NAB = x.shape
    NB = r_ij.shape[2]
    NSB = f_ij.shape[3]
    NF = W1.shape[1]
    BLK_N = N

    b1r = b1.reshape(1, NF)
    b2r = b2.reshape(1, NF)
    boutr = bout.reshape(1, NAB)
    bdr = bd.reshape(1, NAB)
    nbrs = neighbors.astype(jnp.int32)

    grid = (B,)
    full2d = lambda b: (0, 0)
    blk3 = lambda b: (b, 0, 0)

    out = pl.pallas_call(
        functools.partial(_fused_kernel, blk_n=BLK_N, nb=NB,
                          n_atoms=N, nf=NF),
        grid=grid,
        in_specs=[
            pl.BlockSpec((1, N, NAB), blk3),
            pl.BlockSpec((1, BLK_N, NB), blk3),
            pl.BlockSpec((1, BLK_N, NB), blk3),
            pl.BlockSpec((1, BLK_N, NB), blk3),
            pl.BlockSpec((1, BLK_N, NB, NSB), lambda b: (b, 0, 0, 0)),
            pl.BlockSpec((NSB, NF), full2d),
            pl.BlockSpec((1, NF), full2d),
            pl.BlockSpec((NF, NF), full2d),
            pl.BlockSpec((1, NF), full2d),
            pl.BlockSpec((NAB, NF), full2d),
            pl.BlockSpec((NF, NAB), full2d),
            pl.BlockSpec((1, NAB), full2d),
            pl.BlockSpec((NAB, NAB), full2d),
            pl.BlockSpec((1, NAB), full2d),
        ],
        out_specs=pl.BlockSpec((1, BLK_N, NAB), blk3),
        out_shape=jax.ShapeDtypeStruct((B, N, NAB), jnp.float32),
        compiler_params=pltpu.CompilerParams(
            dimension_semantics=("arbitrary",)),
    )(x, r_ij, nbrs, neighbor_mask, f_ij,
      W1, b1r, W2, b2r, Win, Wout, boutr, Wd, bdr)
    return out


# speedup vs baseline: 1.4191x; 1.0558x over previous
import functools
import math

import jax
import jax.numpy as jnp
from jax.experimental import pallas as pl
from jax.experimental.pallas import tpu as pltpu

_CUTOFF = 1.0


def _ssp(v):
    return (jnp.maximum(v, 0.0)
            + jnp.log(0.5 + 0.5 * jnp.exp(-jnp.abs(v))))


def _fused_kernel(x_ref, r_ref, nbr_ref, mask_ref, f_ref,
                  W1_ref, b1_ref, W2_ref, b2_ref, Win_ref,
                  Wout_ref, bout_ref, Wd_ref, bd_ref,
                  o_ref, *, nbatch, nb, n_atoms, nf, nsb):
    W1 = W1_ref[...].astype(jnp.bfloat16)
    W2 = W2_ref[...].astype(jnp.bfloat16)
    Win = Win_ref[...].astype(jnp.bfloat16)

    rows = nbatch * n_atoms * nb
    f = f_ref[...].reshape(rows, nsb).astype(jnp.bfloat16)
    h = _ssp((jnp.dot(f, W1, preferred_element_type=jnp.float32)
              + b1_ref[...]).astype(jnp.bfloat16))
    wf = (jnp.dot(h, W2, preferred_element_type=jnp.float32)
          + b2_ref[...])

    r = r_ref[...]
    z = (0.5 - r * (1.0 / _CUTOFF)) * math.pi
    z2 = z * z
    sinz = z * (1.0 + z2 * (-1.0 / 6.0 + z2 * (1.0 / 120.0 + z2 *
                (-1.0 / 5040.0 + z2 * (1.0 / 362880.0)))))
    c = ((0.5 + 0.5 * sinz)
         * (r < _CUTOFF).astype(jnp.float32) * mask_ref[...])
    c_bf = c.astype(jnp.bfloat16)

    iota = jax.lax.broadcasted_iota(jnp.int16, (n_atoms, nb, n_atoms), 2)
    ynbs = []
    for bb in range(nbatch):
        y = jnp.dot(x_ref[bb].astype(jnp.bfloat16), Win,
                    preferred_element_type=jnp.float32).astype(jnp.bfloat16)
        idx = nbr_ref[bb].astype(jnp.int16)
        oh = jnp.where(idx[:, :, None] == iota, c_bf[bb][:, :, None],
                       jnp.bfloat16(0.0)).reshape(n_atoms * nb, n_atoms)
        ynbs.append(jnp.dot(oh, y, preferred_element_type=jnp.float32))
    ynb = jnp.concatenate(ynbs, axis=0)

    prod = ynb * wf
    agg = jnp.sum(prod.reshape(nbatch * n_atoms, nb, nf), axis=1)

    out = _ssp(jnp.dot(agg, Wout_ref[...], preferred_element_type=jnp.float32)
               + bout_ref[...])
    v = (jnp.dot(out, Wd_ref[...], preferred_element_type=jnp.float32)
         + bd_ref[...])
    o_ref[...] = v.reshape(nbatch, n_atoms, v.shape[-1])


@jax.jit
def kernel(x, r_ij, neighbors, neighbor_mask, f_ij,
           W1, b1, W2, b2, Win, Wout, bout, Wd, bd):
    B, N, NAB = x.shape
    NB = r_ij.shape[2]
    NSB = f_ij.shape[3]
    NF = W1.shape[1]
    NBATCH = 2

    b1r = b1.reshape(1, NF)
    b2r = b2.reshape(1, NF)
    boutr = bout.reshape(1, NAB)
    bdr = bd.reshape(1, NAB)
    nbrs = neighbors.astype(jnp.int32)

    grid = (B // NBATCH,)
    full2d = lambda b: (0, 0)
    blk3 = lambda b: (b, 0, 0)

    out = pl.pallas_call(
        functools.partial(_fused_kernel, nbatch=NBATCH, nb=NB,
                          n_atoms=N, nf=NF, nsb=NSB),
        grid=grid,
        in_specs=[
            pl.BlockSpec((NBATCH, N, NAB), blk3),
            pl.BlockSpec((NBATCH, N, NB), blk3),
            pl.BlockSpec((NBATCH, N, NB), blk3),
            pl.BlockSpec((NBATCH, N, NB), blk3),
            pl.BlockSpec((NBATCH, N, NB, NSB), lambda b: (b, 0, 0, 0)),
            pl.BlockSpec((NSB, NF), full2d),
            pl.BlockSpec((1, NF), full2d),
            pl.BlockSpec((NF, NF), full2d),
            pl.BlockSpec((1, NF), full2d),
            pl.BlockSpec((NAB, NF), full2d),
            pl.BlockSpec((NF, NAB), full2d),
            pl.BlockSpec((1, NAB), full2d),
            pl.BlockSpec((NAB, NAB), full2d),
            pl.BlockSpec((1, NAB), full2d),
        ],
        out_specs=pl.BlockSpec((NBATCH, N, NAB), blk3),
        out_shape=jax.ShapeDtypeStruct((B, N, NAB), jnp.float32),
        compiler_params=pltpu.CompilerParams(
            dimension_semantics=("arbitrary",)),
    )(x, r_ij, nbrs, neighbor_mask, f_ij,
      W1, b1r, W2, b2r, Win, Wout, boutr, Wd, bdr)
    return out


# speedup vs baseline: 1.4241x; 1.0035x over previous
import functools
import math

import jax
import jax.numpy as jnp
from jax.experimental import pallas as pl
from jax.experimental.pallas import tpu as pltpu

_CUTOFF = 1.0


def _ssp(v):
    return (jnp.maximum(v, 0.0)
            + jnp.log(0.5 + 0.5 * jnp.exp(-jnp.abs(v))))


def _fused_kernel(x_ref, r_ref, nbr_ref, mask_ref, f_ref,
                  W1_ref, b1_ref, W2_ref, b2_ref, Win_ref,
                  Wout_ref, bout_ref, Wd_ref, bd_ref,
                  o_ref, *, nbatch, nb, n_atoms, nf, nsb):
    W1 = W1_ref[...].astype(jnp.bfloat16)
    W2 = W2_ref[...].astype(jnp.bfloat16)
    Win = Win_ref[...].astype(jnp.bfloat16)

    rows = nbatch * n_atoms * nb
    f = f_ref[...].reshape(rows, nsb).astype(jnp.bfloat16)
    h = _ssp((jnp.dot(f, W1, preferred_element_type=jnp.float32)
              + b1_ref[...]).astype(jnp.bfloat16))
    wf = (jnp.dot(h, W2, preferred_element_type=jnp.float32)
          + b2_ref[...])

    r = r_ref[...]
    z = (0.5 - r * (1.0 / _CUTOFF)) * math.pi
    z2 = z * z
    sinz = z * (1.0 + z2 * (-1.0 / 6.0 + z2 * (1.0 / 120.0 + z2 *
                (-1.0 / 5040.0 + z2 * (1.0 / 362880.0)))))
    c = ((0.5 + 0.5 * sinz)
         * (r < _CUTOFF).astype(jnp.float32) * mask_ref[...])
    c_bf = c.astype(jnp.bfloat16)

    iota = jax.lax.broadcasted_iota(jnp.int16, (n_atoms, nb, n_atoms), 2)
    ynbs = []
    for bb in range(nbatch):
        y = jnp.dot(x_ref[bb].astype(jnp.bfloat16), Win,
                    preferred_element_type=jnp.float32).astype(jnp.bfloat16)
        idx = nbr_ref[bb].astype(jnp.int16)
        oh = jnp.where(idx[:, :, None] == iota, c_bf[bb][:, :, None],
                       jnp.bfloat16(0.0)).reshape(n_atoms * nb, n_atoms)
        ynbs.append(jnp.dot(oh, y, preferred_element_type=jnp.float32))
    ynb = jnp.concatenate(ynbs, axis=0)

    prod = ynb * wf
    agg = jnp.sum(prod.reshape(nbatch * n_atoms, nb, nf), axis=1)

    out = _ssp(jnp.dot(agg, Wout_ref[...], preferred_element_type=jnp.float32)
               + bout_ref[...])
    v = (jnp.dot(out, Wd_ref[...], preferred_element_type=jnp.float32)
         + bd_ref[...])
    o_ref[...] = v.reshape(nbatch, n_atoms, v.shape[-1])


@jax.jit
def kernel(x, r_ij, neighbors, neighbor_mask, f_ij,
           W1, b1, W2, b2, Win, Wout, bout, Wd, bd):
    B, N, NAB = x.shape
    NB = r_ij.shape[2]
    NSB = f_ij.shape[3]
    NF = W1.shape[1]
    NBATCH = 2

    b1r = b1.reshape(1, NF)
    b2r = b2.reshape(1, NF)
    boutr = bout.reshape(1, NAB)
    bdr = bd.reshape(1, NAB)
    nbrs = neighbors.astype(jnp.int32)

    grid = (B // NBATCH,)
    full2d = lambda b: (0, 0)
    blk3 = lambda b: (b, 0, 0)

    out = pl.pallas_call(
        functools.partial(_fused_kernel, nbatch=NBATCH, nb=NB,
                          n_atoms=N, nf=NF, nsb=NSB),
        grid=grid,
        in_specs=[
            pl.BlockSpec((NBATCH, N, NAB), blk3),
            pl.BlockSpec((NBATCH, N, NB), blk3),
            pl.BlockSpec((NBATCH, N, NB), blk3),
            pl.BlockSpec((NBATCH, N, NB), blk3),
            pl.BlockSpec((NBATCH, N, NB, NSB), lambda b: (b, 0, 0, 0)),
            pl.BlockSpec((NSB, NF), full2d),
            pl.BlockSpec((1, NF), full2d),
            pl.BlockSpec((NF, NF), full2d),
            pl.BlockSpec((1, NF), full2d),
            pl.BlockSpec((NAB, NF), full2d),
            pl.BlockSpec((NF, NAB), full2d),
            pl.BlockSpec((1, NAB), full2d),
            pl.BlockSpec((NAB, NAB), full2d),
            pl.BlockSpec((1, NAB), full2d),
        ],
        out_specs=pl.BlockSpec((NBATCH, N, NAB), blk3),
        out_shape=jax.ShapeDtypeStruct((B, N, NAB), jnp.float32),
        compiler_params=pltpu.CompilerParams(
            dimension_semantics=("parallel",)),
    )(x, r_ij, nbrs, neighbor_mask, f_ij,
      W1, b1r, W2, b2r, Win, Wout, boutr, Wd, bdr)
    return out


# speedup vs baseline: 1.4276x; 1.0025x over previous
import functools
import math

import jax
import jax.numpy as jnp
from jax.experimental import pallas as pl
from jax.experimental.pallas import tpu as pltpu

_CUTOFF = 1.0


def _ssp(v):
    return (jnp.maximum(v, 0.0)
            + jnp.log(0.5 + 0.5 * jnp.exp(-jnp.abs(v))))


def _fused_kernel(x_ref, r_ref, nbr_ref, mask_ref, f_ref,
                  W1_ref, b1_ref, W2_ref, b2_ref, Win_ref,
                  Wout_ref, bout_ref, Wd_ref, bd_ref,
                  o_ref, *, nbatch, nb, n_atoms, nf, nsb):
    W1 = W1_ref[...].astype(jnp.bfloat16)
    W2 = W2_ref[...].astype(jnp.bfloat16)
    Win = Win_ref[...].astype(jnp.bfloat16)

    rows = nbatch * n_atoms * nb
    f = f_ref[...].reshape(rows, nsb).astype(jnp.bfloat16)
    h = _ssp((jnp.dot(f, W1, preferred_element_type=jnp.float32)
              + b1_ref[...]).astype(jnp.bfloat16))
    wf = (jnp.dot(h, W2, preferred_element_type=jnp.float32)
          + b2_ref[...])

    r = r_ref[...]
    z = (0.5 - r * (1.0 / _CUTOFF)) * math.pi
    z2 = z * z
    sinz = z * (1.0 + z2 * (-1.0 / 6.0 + z2 * (1.0 / 120.0 + z2 *
                (-1.0 / 5040.0 + z2 * (1.0 / 362880.0)))))
    c = ((0.5 + 0.5 * sinz)
         * (r < _CUTOFF).astype(jnp.float32) * mask_ref[...])
    c_bf = c.astype(jnp.bfloat16)

    iota = jax.lax.broadcasted_iota(jnp.int16, (n_atoms, nb, n_atoms), 2)
    ynbs = []
    for bb in range(nbatch):
        y = jnp.dot(x_ref[bb].astype(jnp.bfloat16), Win,
                    preferred_element_type=jnp.float32).astype(jnp.bfloat16)
        idx = nbr_ref[bb].astype(jnp.int16)
        oh = jnp.where(idx[:, :, None] == iota, c_bf[bb][:, :, None],
                       jnp.bfloat16(0.0)).reshape(n_atoms * nb, n_atoms)
        ynbs.append(jnp.dot(oh, y, preferred_element_type=jnp.float32))
    ynb = jnp.concatenate(ynbs, axis=0)

    prod = ynb * wf
    agg = jnp.sum(prod.reshape(nbatch * n_atoms, nb, nf), axis=1)

    out = _ssp(jnp.dot(agg, Wout_ref[...], preferred_element_type=jnp.float32)
               + bout_ref[...])
    v = (jnp.dot(out, Wd_ref[...], preferred_element_type=jnp.float32)
         + bd_ref[...])
    o_ref[...] = v.reshape(nbatch, n_atoms, v.shape[-1])


@jax.jit
def kernel(x, r_ij, neighbors, neighbor_mask, f_ij,
           W1, b1, W2, b2, Win, Wout, bout, Wd, bd):
    B, N, NAB = x.shape
    NB = r_ij.shape[2]
    NSB = f_ij.shape[3]
    NF = W1.shape[1]
    NBATCH = 2 if B % 2 == 0 else 1

    b1r = b1.reshape(1, NF)
    b2r = b2.reshape(1, NF)
    boutr = bout.reshape(1, NAB)
    bdr = bd.reshape(1, NAB)
    nbrs = neighbors.astype(jnp.int32)

    grid = (B // NBATCH,)
    full2d = lambda b: (0, 0)
    blk3 = lambda b: (b, 0, 0)

    out = pl.pallas_call(
        functools.partial(_fused_kernel, nbatch=NBATCH, nb=NB,
                          n_atoms=N, nf=NF, nsb=NSB),
        grid=grid,
        in_specs=[
            pl.BlockSpec((NBATCH, N, NAB), blk3),
            pl.BlockSpec((NBATCH, N, NB), blk3),
            pl.BlockSpec((NBATCH, N, NB), blk3),
            pl.BlockSpec((NBATCH, N, NB), blk3),
            pl.BlockSpec((NBATCH, N, NB, NSB), lambda b: (b, 0, 0, 0)),
            pl.BlockSpec((NSB, NF), full2d),
            pl.BlockSpec((1, NF), full2d),
            pl.BlockSpec((NF, NF), full2d),
            pl.BlockSpec((1, NF), full2d),
            pl.BlockSpec((NAB, NF), full2d),
            pl.BlockSpec((NF, NAB), full2d),
            pl.BlockSpec((1, NAB), full2d),
            pl.BlockSpec((NAB, NAB), full2d),
            pl.BlockSpec((1, NAB), full2d),
        ],
        out_specs=pl.BlockSpec((NBATCH, N, NAB), blk3),
        out_shape=jax.ShapeDtypeStruct((B, N, NAB), jnp.float32),
        compiler_params=pltpu.CompilerParams(
            dimension_semantics=("parallel",)),
    )(x, r_ij, nbrs, neighbor_mask, f_ij,
      W1, b1r, W2, b2r, Win, Wout, boutr, Wd, bdr)
    return out
